# baseline (speedup 1.0000x reference)
import jax
import jax.numpy as jnp
from jax import lax
from jax.experimental import pallas as pl
from jax.experimental.pallas import tpu as pltpu
from jax.experimental.pallas import tpu_sc as plsc

B = 128
N = 32768
K = 2048
NV = N // 16
KV = K // 16
UN = 4
MININT = -2147483648
NEG = -3.4028234663852886e38
NEG_HALF = -1.7014117331926443e38
NEG_KEY = 8388608


def _iota16():
  return lax.broadcasted_iota(jnp.int32, (16,), 0)


def _lsr(x, n):
  return lax.shift_right_logical(x, lax.full(x.shape, n, x.dtype))


def _to_key(x):
  s = plsc.bitcast(x, jnp.int32)
  return jnp.where(s >= 0, s | jnp.int32(MININT), ~s)


def _from_key(key):
  bits = jnp.where(key < 0, key & jnp.int32(0x7FFFFFFF), ~key)
  return plsc.bitcast(bits, jnp.float32)


def _clear(ref, nbins):
  for v in range(nbins // 16):
    ref[pl.ds(v * 16, 16)] = jnp.zeros((16,), jnp.int32)


def _merge_hists(hist4, hist):
  for v in range(16):
    s = hist4[pl.ds(v * 16, 16)]
    for j in range(1, UN):
      s = s + hist4[pl.ds(j * 256 + v * 16, 16)]
    hist[pl.ds(v * 16, 16)] = s


def _select_bucket(hist, sufs, kneed):
  run = jnp.int32(0)
  bstar = jnp.int32(-1)
  for v in range(15, -1, -1):
    x = hist[pl.ds(v * 16, 16)]
    c = plsc.cumsum(x)
    tot = jnp.max(c)
    suf = (tot - c) + x + run
    sufs[pl.ds(v * 16, 16)] = suf
    bid = _iota16() + jnp.int32(v * 16)
    cand = jnp.where(suf >= kneed, bid, jnp.int32(-1))
    bstar = jnp.maximum(bstar, jnp.max(cand))
    run = run + tot
  bvec = jnp.broadcast_to(bstar, (16,))
  s_b = jnp.max(plsc.load_gather(sufs, [bvec]))
  h_b = jnp.max(plsc.load_gather(hist, [bvec]))
  return bstar, s_b - h_b, s_b, h_b


def _sc_body(scores_hbm, lengths_hbm, vals_hbm, idx_hbm,
             row_f, buf_k, buf_i, acc_k, acc_i, tmp_k, tmp_i,
             hist4, hists16, hist, sufs, offs_a, offs_b, offs_c, offs_d,
             len_v, vals_out, idx_out, dma_sem):
  cid = lax.axis_index("c")
  sid = lax.axis_index("s")
  wid = cid * 16 + sid
  pltpu.sync_copy(lengths_hbm, len_v)
  pltpu.async_copy(scores_hbm.at[pl.ds(wid * N, N)], row_f, dma_sem)

  def row_body(r, _):
    row = wid + 32 * r
    grp16 = (row >> 4) << 4
    lv = len_v[pl.ds(grp16, 16)]
    lane = row - grp16
    length = jnp.max(jnp.where(_iota16() == lane, lv, jnp.int32(0)))
    nvl4 = ((length + 15) >> 4).astype(jnp.int32)
    nvl4 = (nvl4 + (UN - 1)) >> 2
    tail_start = nvl4 * (UN * 16)
    tail_cnt = jnp.int32(N) - tail_start

    pltpu.make_async_copy(scores_hbm.at[pl.ds(row * N, N)], row_f,
                          dma_sem).wait()

    def clr_body(i, _):
      for j in range(UN):
        hists16[pl.ds((i * UN + j) * 16, 16)] = jnp.zeros((16,), jnp.int32)
      return 0

    lax.fori_loop(0, 256 // UN, clr_body, 0)
    ones = jnp.ones((16,), jnp.int32)

    @plsc.parallel_loop(0, nvl4 * UN, 1, unroll=UN)
    def a_body(i):
      b = i * 16
      key = _to_key(row_f[pl.ds(b, 16)])
      pos = _iota16() + b
      key = jnp.where(pos < length, key, jnp.int32(NEG_KEY))
      d = _lsr(key, 24)
      addr = (d << 4) | _iota16()
      plsc.addupdate_scatter(hists16, [addr], ones)
    lane16 = _iota16() * 16
    for v in range(16):
      tot = plsc.load_gather(hists16, [lane16 + jnp.int32(v * 256)])
      for l in range(1, 16):
        tot = tot + plsc.load_gather(hists16,
                                     [lane16 + jnp.int32(v * 256 + l)])
      hist[pl.ds(v * 16, 16)] = tot
    h0 = hist[pl.ds(0, 16)]
    hist[pl.ds(0, 16)] = h0 + jnp.where(_iota16() == 0, tail_cnt,
                                        jnp.int32(0))

    kneed = jnp.int32(K)
    bstar, g_above, s_b, h_b = _select_bucket(hist, sufs, kneed)

    def c_body(i, carry):
      coff = carry
      keys, cms, ccs, poss = [], [], [], []
      for j in range(UN):
        b = (i * UN + j) * 16
        key = _to_key(row_f[pl.ds(b, 16)])
        pos = _iota16() + b
        key = jnp.where(pos < length, key, jnp.int32(NEG_KEY))
        d = _lsr(key, 24)
        cm = d >= bstar
        keys.append(key)
        poss.append(pos)
        cms.append(cm)
        ccs.append(jnp.sum(cm.astype(jnp.int32)))
      for j in range(UN):
        plsc.store_compressed(buf_k.at[pl.ds(coff, 16)], keys[j],
                              mask=cms[j])
        plsc.store_compressed(buf_i.at[pl.ds(coff, 16)], poss[j],
                              mask=cms[j])
        coff = coff + ccs[j]
      return coff

    coff = lax.fori_loop(0, nvl4, c_body, jnp.int32(0))
    kneed = kneed - g_above

    tail_stop = jnp.minimum(jnp.int32(NV),
                            (tail_start >> 4) + ((kneed + 15) >> 4))
    nact = jnp.where(bstar == 0, coff + (tail_stop << 4) - tail_start, s_b)

    @pl.when(bstar == 0)
    def _():
      all_true = jnp.ones((16,), jnp.bool_)
      negs = jnp.full((16,), NEG_KEY, jnp.int32)

      def t_body(v, coff_t):
        plsc.store_compressed(buf_k.at[pl.ds(coff_t, 16)], negs,
                              mask=all_true)
        plsc.store_compressed(buf_i.at[pl.ds(coff_t, 16)],
                              _iota16() + v * 16, mask=all_true)
        return coff_t + 16

      lax.fori_loop(tail_start >> 4, tail_stop, t_body, coff)

    @pl.when(r < 3)
    def _():
      pltpu.async_copy(scores_hbm.at[pl.ds((row + 32) * N, N)], row_f,
                       dma_sem)

    prefix = bstar

    def level(shift, prefix, kneed, nact):
      _clear(hist4, UN * 256)
      nv = (nact + 15) >> 4

      @plsc.parallel_loop(0, nv, 1, unroll=UN)
      def h_body(i):
        b = i * 16
        key = buf_k[pl.ds(b, 16)]
        cand = (_lsr(key, shift + 8) == prefix) & ((_iota16() + b) < nact)
        d = _lsr(key, shift) & jnp.int32(0xFF)
        occ, last = plsc.scan_count(d, mask=cand)
        plsc.addupdate_scatter(hist4, [d + (i & 3) * 256], occ, mask=last)

      _merge_hists(hist4, hist)
      bs, g_above, _sb, _hb = _select_bucket(hist, sufs, kneed)

      def d_body(i, carry):
        noff = carry
        keys, ivs, kms, kcs = [], [], [], []
        for j in range(UN):
          b = (i * UN + j) * 16
          key = buf_k[pl.ds(b, 16)]
          iv = buf_i[pl.ds(b, 16)]
          lvm = (_iota16() + b) < nact
          cand = _lsr(key, shift + 8) == prefix
          d = _lsr(key, shift) & jnp.int32(0xFF)
          km = ((~cand) | (d >= bs)) & lvm
          keys.append(key)
          ivs.append(iv)
          kms.append(km)
          kcs.append(jnp.sum(km.astype(jnp.int32)))
        for j in range(UN):
          plsc.store_compressed(buf_k.at[pl.ds(noff, 16)], keys[j],
                                mask=kms[j])
          plsc.store_compressed(buf_i.at[pl.ds(noff, 16)], ivs[j],
                                mask=kms[j])
          noff = noff + kcs[j]
        return noff

      nact = lax.fori_loop(0, (nv + UN - 1) >> 2, d_body, jnp.int32(0))
      return (prefix << 8) | bs, kneed - g_above, nact

    prefix, kneed, nact = level(16, prefix, kneed, nact)
    prefix, kneed, nact = level(8, prefix, kneed, nact)
    prefix, kneed, nact = level(0, prefix, kneed, nact)

    def e_body(i, carry):
      aoff, run_eq = carry
      base = i * 16
      key = buf_k[pl.ds(base, 16)]
      iv = buf_i[pl.ds(base, 16)]
      valid = (_iota16() + base) < nact
      eq = (key == prefix) & valid
      pe = plsc.cumsum(eq.astype(jnp.int32))
      keep = valid & ((~eq) | ((pe + run_eq) <= kneed))
      plsc.store_compressed(acc_k.at[pl.ds(aoff, 16)], key, mask=keep)
      plsc.store_compressed(acc_i.at[pl.ds(aoff, 16)], iv, mask=keep)
      return (aoff + jnp.sum(keep.astype(jnp.int32)), run_eq + jnp.max(pe))

    lax.fori_loop(0, (nact + 15) >> 4, e_body,
                  (jnp.int32(0), jnp.int32(0)))

    def sort_pass(src_k, src_i, dst_k, dst_i, shift, invert):
      _clear(hist4, UN * 256)
      qoffs = [offs_a, offs_b, offs_c, offs_d]

      @plsc.parallel_loop(0, KV, 1, unroll=UN)
      def h_body(i):
        k_ = src_k[pl.ds(i * 16, 16)]
        if invert:
          k_ = ~k_
        d = _lsr(k_, shift) & jnp.int32(0xFF)
        occ, last = plsc.scan_count(d)
        plsc.addupdate_scatter(hist4, [d + (i >> 5) * 256], occ, mask=last)

      run = jnp.int32(0)
      for v in range(16):
        hq = [hist4[pl.ds(q * 256 + v * 16, 16)] for q in range(4)]
        tot = (hq[0] + hq[1]) + (hq[2] + hq[3])
        c = plsc.cumsum(tot)
        ex = c - tot + run
        qoffs[0][pl.ds(v * 16, 16)] = ex
        ex = ex + hq[0]
        qoffs[1][pl.ds(v * 16, 16)] = ex
        ex = ex + hq[1]
        qoffs[2][pl.ds(v * 16, 16)] = ex
        ex = ex + hq[2]
        qoffs[3][pl.ds(v * 16, 16)] = ex
        run = run + jnp.max(c)

      def s_body(i, _):
        for q in range(4):
          b = (q * 32 + i) * 16
          k_ = src_k[pl.ds(b, 16)]
          iv = src_i[pl.ds(b, 16)]
          if invert:
            k_ = ~k_
          d = _lsr(k_, shift) & jnp.int32(0xFF)
          occ, last = plsc.scan_count(d)
          basev = plsc.load_gather(qoffs[q], [d])
          posn = basev + occ - 1
          plsc.store_scatter(dst_k, [posn], k_)
          plsc.store_scatter(dst_i, [posn], iv)
          plsc.addupdate_scatter(qoffs[q], [d], occ, mask=last)
        return 0

      lax.fori_loop(0, KV // 4, s_body, 0)

    sort_pass(acc_k, acc_i, tmp_k, tmp_i, 0, True)
    sort_pass(tmp_k, tmp_i, acc_k, acc_i, 8, False)
    sort_pass(acc_k, acc_i, tmp_k, tmp_i, 16, False)
    sort_pass(tmp_k, tmp_i, acc_k, acc_i, 24, False)

    @plsc.parallel_loop(0, KV, 1, unroll=UN)
    def g_body(i):
      b = i * 16
      sk = acc_k[pl.ds(b, 16)]
      v = _from_key(~sk)
      vals_out[pl.ds(b, 16)] = v
      iv = acc_i[pl.ds(b, 16)]
      idx_out[pl.ds(b, 16)] = jnp.where(
          v > jnp.float32(NEG_HALF), iv, jnp.int32(-1))
    pltpu.sync_copy(vals_out, vals_hbm.at[pl.ds(row * K, K)])
    pltpu.sync_copy(idx_out, idx_hbm.at[pl.ds(row * K, K)])
    return 0

  lax.fori_loop(0, 4, row_body, 0)


@jax.jit
def _sc_topk(scores_flat, lengths):
  mesh = plsc.VectorSubcoreMesh(core_axis_name="c", subcore_axis_name="s")
  f = pl.kernel(
      _sc_body,
      out_type=(jax.ShapeDtypeStruct((B * K,), jnp.float32),
                jax.ShapeDtypeStruct((B * K,), jnp.int32)),
      mesh=mesh,
      compiler_params=pltpu.CompilerParams(needs_layout_passes=False),
      scratch_types=[
          pltpu.VMEM((N,), jnp.float32),
          pltpu.VMEM((N + 16,), jnp.int32),
          pltpu.VMEM((N + 16,), jnp.int32),
          pltpu.VMEM((K + 16,), jnp.int32),
          pltpu.VMEM((K + 16,), jnp.int32),
          pltpu.VMEM((K,), jnp.int32),
          pltpu.VMEM((K,), jnp.int32),
          pltpu.VMEM((UN * 256,), jnp.int32),
          pltpu.VMEM((4096,), jnp.int32),
          pltpu.VMEM((256,), jnp.int32),
          pltpu.VMEM((256,), jnp.int32),
          pltpu.VMEM((256,), jnp.int32),
          pltpu.VMEM((256,), jnp.int32),
          pltpu.VMEM((256,), jnp.int32),
          pltpu.VMEM((256,), jnp.int32),
          pltpu.VMEM((B,), jnp.int32),
          pltpu.VMEM((K,), jnp.float32),
          pltpu.VMEM((K,), jnp.int32),
          pltpu.SemaphoreType.DMA,
      ],
  )
  return f(scores_flat, lengths)


def kernel(scores, lengths, k):
  del k
  vals_flat, idx_flat = _sc_topk(scores.reshape(-1), lengths)
  return vals_flat.reshape(B, K), idx_flat.reshape(B, K)


# speedup vs baseline: 1.0694x; 1.0694x over previous
import jax
import jax.numpy as jnp
from jax import lax
from jax.experimental import pallas as pl
from jax.experimental.pallas import tpu as pltpu
from jax.experimental.pallas import tpu_sc as plsc

B = 128
N = 32768
K = 2048
NV = N // 16
KV = K // 16
UN = 4
MININT = -2147483648
NEG = -3.4028234663852886e38
NEG_HALF = -1.7014117331926443e38
NEG_KEY = 8388608


def _iota16():
  return lax.broadcasted_iota(jnp.int32, (16,), 0)


def _lsr(x, n):
  return lax.shift_right_logical(x, lax.full(x.shape, n, x.dtype))


def _to_key(x):
  s = plsc.bitcast(x, jnp.int32)
  return jnp.where(s >= 0, s | jnp.int32(MININT), ~s)


def _from_key(key):
  bits = jnp.where(key < 0, key & jnp.int32(0x7FFFFFFF), ~key)
  return plsc.bitcast(bits, jnp.float32)


def _clear(ref, nbins):
  for v in range(nbins // 16):
    ref[pl.ds(v * 16, 16)] = jnp.zeros((16,), jnp.int32)


def _merge_hists(hist4, hist):
  for v in range(16):
    s = hist4[pl.ds(v * 16, 16)]
    for j in range(1, UN):
      s = s + hist4[pl.ds(j * 256 + v * 16, 16)]
    hist[pl.ds(v * 16, 16)] = s


def _select_bucket(hist, sufs, kneed):
  run = jnp.int32(0)
  bstar = jnp.int32(-1)
  for v in range(15, -1, -1):
    x = hist[pl.ds(v * 16, 16)]
    c = plsc.cumsum(x)
    tot = jnp.max(c)
    suf = (tot - c) + x + run
    sufs[pl.ds(v * 16, 16)] = suf
    bid = _iota16() + jnp.int32(v * 16)
    cand = jnp.where(suf >= kneed, bid, jnp.int32(-1))
    bstar = jnp.maximum(bstar, jnp.max(cand))
    run = run + tot
  bvec = jnp.broadcast_to(bstar, (16,))
  s_b = jnp.max(plsc.load_gather(sufs, [bvec]))
  h_b = jnp.max(plsc.load_gather(hist, [bvec]))
  return bstar, s_b - h_b, h_b


def _sc_body(scores_hbm, lengths_hbm, vals_hbm, idx_hbm,
             row_f, buf_k, buf_i, acc_k, acc_i, tmp_k, tmp_i,
             hist4, hists16, hist, sufs, offs_a, offs_b, offs_c, offs_d,
             len_v, vals_out, idx_out, dma_sem):
  cid = lax.axis_index("c")
  sid = lax.axis_index("s")
  wid = cid * 16 + sid
  pltpu.sync_copy(lengths_hbm, len_v)
  pltpu.async_copy(scores_hbm.at[pl.ds(wid * N, N)], row_f, dma_sem)

  def row_body(r, _):
    row = wid + 32 * r
    grp16 = (row >> 4) << 4
    lv = len_v[pl.ds(grp16, 16)]
    lane = row - grp16
    length = jnp.max(jnp.where(_iota16() == lane, lv, jnp.int32(0)))
    nvl4 = ((length + 15) >> 4).astype(jnp.int32)
    nvl4 = (nvl4 + (UN - 1)) >> 2
    tail_start = nvl4 * (UN * 16)
    tail_cnt = jnp.int32(N) - tail_start

    pltpu.make_async_copy(scores_hbm.at[pl.ds(row * N, N)], row_f,
                          dma_sem).wait()

    def clr_body(i, _):
      for j in range(UN):
        hists16[pl.ds((i * UN + j) * 16, 16)] = jnp.zeros((16,), jnp.int32)
      return 0

    lax.fori_loop(0, 256 // UN, clr_body, 0)
    ones = jnp.ones((16,), jnp.int32)

    @plsc.parallel_loop(0, nvl4 * UN, 1, unroll=UN)
    def a_body(i):
      b = i * 16
      key = _to_key(row_f[pl.ds(b, 16)])
      pos = _iota16() + b
      key = jnp.where(pos < length, key, jnp.int32(NEG_KEY))
      d = _lsr(key, 24)
      addr = (d << 4) | _iota16()
      plsc.addupdate_scatter(hists16, [addr], ones)
    lane16 = _iota16() * 16
    for v in range(16):
      tot = plsc.load_gather(hists16, [lane16 + jnp.int32(v * 256)])
      for l in range(1, 16):
        tot = tot + plsc.load_gather(hists16,
                                     [lane16 + jnp.int32(v * 256 + l)])
      hist[pl.ds(v * 16, 16)] = tot
    h0 = hist[pl.ds(0, 16)]
    hist[pl.ds(0, 16)] = h0 + jnp.where(_iota16() == 0, tail_cnt,
                                        jnp.int32(0))

    kneed = jnp.int32(K)
    bstar, g_above, h_b = _select_bucket(hist, sufs, kneed)

    def c_body(i, carry):
      aoff, coff = carry
      keys, ams, cms, cas, ccs, poss = [], [], [], [], [], []
      for j in range(UN):
        b = (i * UN + j) * 16
        key = _to_key(row_f[pl.ds(b, 16)])
        pos = _iota16() + b
        key = jnp.where(pos < length, key, jnp.int32(NEG_KEY))
        d = _lsr(key, 24)
        am = d > bstar
        cm = d == bstar
        keys.append(key)
        poss.append(pos)
        ams.append(am)
        cms.append(cm)
        cas.append(jnp.sum(am.astype(jnp.int32)))
        ccs.append(jnp.sum(cm.astype(jnp.int32)))
      for j in range(UN):
        plsc.store_compressed(acc_k.at[pl.ds(aoff, 16)], keys[j],
                              mask=ams[j])
        plsc.store_compressed(acc_i.at[pl.ds(aoff, 16)], poss[j],
                              mask=ams[j])
        plsc.store_compressed(buf_k.at[pl.ds(coff, 16)], keys[j],
                              mask=cms[j])
        plsc.store_compressed(buf_i.at[pl.ds(coff, 16)], poss[j],
                              mask=cms[j])
        aoff = aoff + cas[j]
        coff = coff + ccs[j]
      return (aoff, coff)

    aoff, coff = lax.fori_loop(0, nvl4, c_body,
                               (jnp.int32(0), jnp.int32(0)))

    kneed = kneed - g_above

    tail_stop = jnp.minimum(jnp.int32(NV),
                            (tail_start >> 4) + ((kneed + 15) >> 4))
    ncand = jnp.where(bstar == 0, coff + (tail_stop << 4) - tail_start, h_b)

    @pl.when(bstar == 0)
    def _():
      all_true = jnp.ones((16,), jnp.bool_)
      negs = jnp.full((16,), NEG_KEY, jnp.int32)

      def t_body(v, coff_t):
        plsc.store_compressed(buf_k.at[pl.ds(coff_t, 16)], negs,
                              mask=all_true)
        plsc.store_compressed(buf_i.at[pl.ds(coff_t, 16)],
                              _iota16() + v * 16, mask=all_true)
        return coff_t + 16

      lax.fori_loop(tail_start >> 4, tail_stop, t_body, coff)

    @pl.when(r < 3)
    def _():
      pltpu.async_copy(scores_hbm.at[pl.ds((row + 32) * N, N)], row_f,
                       dma_sem)

    count = ncand

    def level(shift, aoff, kneed, count):
      _clear(hist4, UN * 256)
      nv4 = (count + (UN * 16 - 1)) >> 6

      @plsc.parallel_loop(0, nv4 * UN, 1, unroll=UN)
      def h_body(i):
        b = i * 16
        key = buf_k[pl.ds(b, 16)]
        lvm = (_iota16() + b) < count
        d = _lsr(key, shift) & jnp.int32(0xFF)
        occ, last = plsc.scan_count(d, mask=lvm)
        plsc.addupdate_scatter(hist4, [d + (i & 3) * 256], occ, mask=last)

      _merge_hists(hist4, hist)
      bstar, g_above, h_b = _select_bucket(hist, sufs, kneed)

      def d_body(i, carry):
        aoff, coff = carry
        keys, ivs, ams, cms, cas, ccs = [], [], [], [], [], []
        for j in range(UN):
          b = (i * UN + j) * 16
          key = buf_k[pl.ds(b, 16)]
          iv = buf_i[pl.ds(b, 16)]
          lvm = (_iota16() + b) < count
          d = _lsr(key, shift) & jnp.int32(0xFF)
          am = (d > bstar) & lvm
          cm = (d == bstar) & lvm
          keys.append(key)
          ivs.append(iv)
          ams.append(am)
          cms.append(cm)
          cas.append(jnp.sum(am.astype(jnp.int32)))
          ccs.append(jnp.sum(cm.astype(jnp.int32)))
        for j in range(UN):
          plsc.store_compressed(acc_k.at[pl.ds(aoff, 16)], keys[j],
                                mask=ams[j])
          plsc.store_compressed(acc_i.at[pl.ds(aoff, 16)], ivs[j],
                                mask=ams[j])
          plsc.store_compressed(buf_k.at[pl.ds(coff, 16)], keys[j],
                                mask=cms[j])
          plsc.store_compressed(buf_i.at[pl.ds(coff, 16)], ivs[j],
                                mask=cms[j])
          aoff = aoff + cas[j]
          coff = coff + ccs[j]
        return (aoff, coff)

      aoff, _u = lax.fori_loop(0, nv4, d_body, (aoff, jnp.int32(0)))
      return aoff, kneed - g_above, h_b

    aoff, kneed, count = level(16, aoff, kneed, count)
    aoff, kneed, count = level(8, aoff, kneed, count)
    aoff, kneed, count = level(0, aoff, kneed, count)

    def e_body(i, aoff):
      base = i * 16
      key = buf_k[pl.ds(base, 16)]
      iv = buf_i[pl.ds(base, 16)]
      m = (_iota16() + base) < kneed
      plsc.store_compressed(acc_k.at[pl.ds(aoff, 16)], key, mask=m)
      plsc.store_compressed(acc_i.at[pl.ds(aoff, 16)], iv, mask=m)
      return aoff + jnp.sum(m.astype(jnp.int32))

    lax.fori_loop(0, (kneed + 15) >> 4, e_body, aoff)

    def sort_pass(src_k, src_i, dst_k, dst_i, shift, invert):
      _clear(hist4, UN * 256)
      qoffs = [offs_a, offs_b, offs_c, offs_d]

      @plsc.parallel_loop(0, KV, 1, unroll=UN)
      def h_body(i):
        k_ = src_k[pl.ds(i * 16, 16)]
        if invert:
          k_ = ~k_
        d = _lsr(k_, shift) & jnp.int32(0xFF)
        occ, last = plsc.scan_count(d)
        plsc.addupdate_scatter(hist4, [d + (i >> 5) * 256], occ, mask=last)

      run = jnp.int32(0)
      for v in range(16):
        hq = [hist4[pl.ds(q * 256 + v * 16, 16)] for q in range(4)]
        tot = (hq[0] + hq[1]) + (hq[2] + hq[3])
        c = plsc.cumsum(tot)
        ex = c - tot + run
        qoffs[0][pl.ds(v * 16, 16)] = ex
        ex = ex + hq[0]
        qoffs[1][pl.ds(v * 16, 16)] = ex
        ex = ex + hq[1]
        qoffs[2][pl.ds(v * 16, 16)] = ex
        ex = ex + hq[2]
        qoffs[3][pl.ds(v * 16, 16)] = ex
        run = run + jnp.max(c)

      def s_body(i, _):
        for q in range(4):
          b = (q * 32 + i) * 16
          k_ = src_k[pl.ds(b, 16)]
          iv = src_i[pl.ds(b, 16)]
          if invert:
            k_ = ~k_
          d = _lsr(k_, shift) & jnp.int32(0xFF)
          occ, last = plsc.scan_count(d)
          basev = plsc.load_gather(qoffs[q], [d])
          posn = basev + occ - 1
          plsc.store_scatter(dst_k, [posn], k_)
          plsc.store_scatter(dst_i, [posn], iv)
          plsc.addupdate_scatter(qoffs[q], [d], occ, mask=last)
        return 0

      lax.fori_loop(0, KV // 4, s_body, 0)

    sort_pass(acc_k, acc_i, tmp_k, tmp_i, 0, True)
    sort_pass(tmp_k, tmp_i, acc_k, acc_i, 8, False)
    sort_pass(acc_k, acc_i, tmp_k, tmp_i, 16, False)
    sort_pass(tmp_k, tmp_i, acc_k, acc_i, 24, False)

    @plsc.parallel_loop(0, KV, 1, unroll=UN)
    def g_body(i):
      b = i * 16
      sk = acc_k[pl.ds(b, 16)]
      v = _from_key(~sk)
      vals_out[pl.ds(b, 16)] = v
      iv = acc_i[pl.ds(b, 16)]
      idx_out[pl.ds(b, 16)] = jnp.where(
          v > jnp.float32(NEG_HALF), iv, jnp.int32(-1))
    pltpu.sync_copy(vals_out, vals_hbm.at[pl.ds(row * K, K)])
    pltpu.sync_copy(idx_out, idx_hbm.at[pl.ds(row * K, K)])
    return 0

  lax.fori_loop(0, 4, row_body, 0)


@jax.jit
def _sc_topk(scores_flat, lengths):
  mesh = plsc.VectorSubcoreMesh(core_axis_name="c", subcore_axis_name="s")
  f = pl.kernel(
      _sc_body,
      out_type=(jax.ShapeDtypeStruct((B * K,), jnp.float32),
                jax.ShapeDtypeStruct((B * K,), jnp.int32)),
      mesh=mesh,
      compiler_params=pltpu.CompilerParams(needs_layout_passes=False),
      scratch_types=[
          pltpu.VMEM((N,), jnp.float32),
          pltpu.VMEM((N + 16,), jnp.int32),
          pltpu.VMEM((N + 16,), jnp.int32),
          pltpu.VMEM((K + 16,), jnp.int32),
          pltpu.VMEM((K + 16,), jnp.int32),
          pltpu.VMEM((K,), jnp.int32),
          pltpu.VMEM((K,), jnp.int32),
          pltpu.VMEM((UN * 256,), jnp.int32),
          pltpu.VMEM((4096,), jnp.int32),
          pltpu.VMEM((256,), jnp.int32),
          pltpu.VMEM((256,), jnp.int32),
          pltpu.VMEM((256,), jnp.int32),
          pltpu.VMEM((256,), jnp.int32),
          pltpu.VMEM((256,), jnp.int32),
          pltpu.VMEM((256,), jnp.int32),
          pltpu.VMEM((B,), jnp.int32),
          pltpu.VMEM((K,), jnp.float32),
          pltpu.VMEM((K,), jnp.int32),
          pltpu.SemaphoreType.DMA,
      ],
  )
  return f(scores_flat, lengths)


def kernel(scores, lengths, k):
  del k
  vals_flat, idx_flat = _sc_topk(scores.reshape(-1), lengths)
  return vals_flat.reshape(B, K), idx_flat.reshape(B, K)


# speedup vs baseline: 1.2200x; 1.1408x over previous
import jax
import jax.numpy as jnp
from jax import lax
from jax.experimental import pallas as pl
from jax.experimental.pallas import tpu as pltpu
from jax.experimental.pallas import tpu_sc as plsc

B = 128
N = 32768
K = 2048
NV = N // 16
KV = K // 16
UN = 4
MININT = -2147483648
NEG = -3.4028234663852886e38
NEG_HALF = -1.7014117331926443e38
NEG_KEY = 8388608


def _iota16():
  return lax.broadcasted_iota(jnp.int32, (16,), 0)


def _lsr(x, n):
  return lax.shift_right_logical(x, lax.full(x.shape, n, x.dtype))


def _to_key(x):
  s = plsc.bitcast(x, jnp.int32)
  return jnp.where(s >= 0, s | jnp.int32(MININT), ~s)


def _from_key(key):
  bits = jnp.where(key < 0, key & jnp.int32(0x7FFFFFFF), ~key)
  return plsc.bitcast(bits, jnp.float32)


def _clear(ref, nbins):
  for v in range(nbins // 16):
    ref[pl.ds(v * 16, 16)] = jnp.zeros((16,), jnp.int32)


def _merge_hists(hist4, hist):
  for v in range(16):
    s = hist4[pl.ds(v * 16, 16)]
    for j in range(1, UN):
      s = s + hist4[pl.ds(j * 256 + v * 16, 16)]
    hist[pl.ds(v * 16, 16)] = s


def _select_bucket(hist, sufs, kneed):
  run = jnp.int32(0)
  bstar = jnp.int32(-1)
  for v in range(15, -1, -1):
    x = hist[pl.ds(v * 16, 16)]
    c = plsc.cumsum(x)
    tot = jnp.max(c)
    suf = (tot - c) + x + run
    sufs[pl.ds(v * 16, 16)] = suf
    bid = _iota16() + jnp.int32(v * 16)
    cand = jnp.where(suf >= kneed, bid, jnp.int32(-1))
    bstar = jnp.maximum(bstar, jnp.max(cand))
    run = run + tot
  bvec = jnp.broadcast_to(bstar, (16,))
  s_b = jnp.max(plsc.load_gather(sufs, [bvec]))
  h_b = jnp.max(plsc.load_gather(hist, [bvec]))
  return bstar, s_b - h_b, h_b


def _sc_body(scores_hbm, lengths_hbm, vals_hbm, idx_hbm,
             row_f, buf_k, buf_i, acc_k, acc_i, tmp_k, tmp_i,
             hist4, hists16, hist, sufs, offs_a, offs_b, offs_c, offs_d,
             len_v, vals_out, idx_out, dma_sem):
  cid = lax.axis_index("c")
  sid = lax.axis_index("s")
  wid = cid * 16 + sid
  pltpu.sync_copy(lengths_hbm, len_v)
  pltpu.async_copy(scores_hbm.at[wid], row_f, dma_sem)

  def row_body(r, _):
    row = wid + 32 * r
    grp16 = (row >> 4) << 4
    lv = len_v[pl.ds(grp16, 16)]
    lane = row - grp16
    length = jnp.max(jnp.where(_iota16() == lane, lv, jnp.int32(0)))
    nvl4 = ((length + 15) >> 4).astype(jnp.int32)
    nvl4 = (nvl4 + (UN - 1)) >> 2
    tail_start = nvl4 * (UN * 16)
    tail_cnt = jnp.int32(N) - tail_start

    pltpu.make_async_copy(scores_hbm.at[row], row_f,
                          dma_sem).wait()

    def clr_body(i, _):
      for j in range(UN):
        hists16[pl.ds((i * UN + j) * 16, 16)] = jnp.zeros((16,), jnp.int32)
      return 0

    lax.fori_loop(0, 256 // UN, clr_body, 0)
    ones = jnp.ones((16,), jnp.int32)

    @plsc.parallel_loop(0, nvl4 * UN, 1, unroll=UN)
    def a_body(i):
      b = i * 16
      key = _to_key(row_f[pl.ds(b, 16)])
      pos = _iota16() + b
      key = jnp.where(pos < length, key, jnp.int32(NEG_KEY))
      d = _lsr(key, 24)
      addr = (d << 4) | _iota16()
      plsc.addupdate_scatter(hists16, [addr], ones)
    lane16 = _iota16() * 16
    for v in range(16):
      tot = plsc.load_gather(hists16, [lane16 + jnp.int32(v * 256)])
      for l in range(1, 16):
        tot = tot + plsc.load_gather(hists16,
                                     [lane16 + jnp.int32(v * 256 + l)])
      hist[pl.ds(v * 16, 16)] = tot
    h0 = hist[pl.ds(0, 16)]
    hist[pl.ds(0, 16)] = h0 + jnp.where(_iota16() == 0, tail_cnt,
                                        jnp.int32(0))

    kneed = jnp.int32(K)
    bstar, g_above, h_b = _select_bucket(hist, sufs, kneed)

    def c_body(i, carry):
      aoff, coff = carry
      keys, ams, cms, cas, ccs, poss = [], [], [], [], [], []
      for j in range(UN):
        b = (i * UN + j) * 16
        key = _to_key(row_f[pl.ds(b, 16)])
        pos = _iota16() + b
        key = jnp.where(pos < length, key, jnp.int32(NEG_KEY))
        d = _lsr(key, 24)
        am = d > bstar
        cm = d == bstar
        keys.append(key)
        poss.append(pos)
        ams.append(am)
        cms.append(cm)
        cas.append(jnp.sum(am.astype(jnp.int32)))
        ccs.append(jnp.sum(cm.astype(jnp.int32)))
      for j in range(UN):
        plsc.store_compressed(acc_k.at[pl.ds(aoff, 16)], keys[j],
                              mask=ams[j])
        plsc.store_compressed(acc_i.at[pl.ds(aoff, 16)], poss[j],
                              mask=ams[j])
        plsc.store_compressed(buf_k.at[pl.ds(coff, 16)], keys[j],
                              mask=cms[j])
        plsc.store_compressed(buf_i.at[pl.ds(coff, 16)], poss[j],
                              mask=cms[j])
        aoff = aoff + cas[j]
        coff = coff + ccs[j]
      return (aoff, coff)

    aoff, coff = lax.fori_loop(0, nvl4, c_body,
                               (jnp.int32(0), jnp.int32(0)))

    kneed = kneed - g_above

    tail_stop = jnp.minimum(jnp.int32(NV),
                            (tail_start >> 4) + ((kneed + 15) >> 4))
    ncand = jnp.where(bstar == 0, coff + (tail_stop << 4) - tail_start, h_b)

    @pl.when(bstar == 0)
    def _():
      all_true = jnp.ones((16,), jnp.bool_)
      negs = jnp.full((16,), NEG_KEY, jnp.int32)

      def t_body(v, coff_t):
        plsc.store_compressed(buf_k.at[pl.ds(coff_t, 16)], negs,
                              mask=all_true)
        plsc.store_compressed(buf_i.at[pl.ds(coff_t, 16)],
                              _iota16() + v * 16, mask=all_true)
        return coff_t + 16

      lax.fori_loop(tail_start >> 4, tail_stop, t_body, coff)

    @pl.when(r < 3)
    def _():
      pltpu.async_copy(scores_hbm.at[row + 32], row_f,
                       dma_sem)

    count = ncand

    def level(shift, aoff, kneed, count):
      _clear(hist4, UN * 256)
      nv4 = (count + (UN * 16 - 1)) >> 6

      @plsc.parallel_loop(0, nv4 * UN, 1, unroll=UN)
      def h_body(i):
        b = i * 16
        key = buf_k[pl.ds(b, 16)]
        lvm = (_iota16() + b) < count
        d = _lsr(key, shift) & jnp.int32(0xFF)
        occ, last = plsc.scan_count(d, mask=lvm)
        plsc.addupdate_scatter(hist4, [d + (i & 3) * 256], occ, mask=last)

      _merge_hists(hist4, hist)
      bstar, g_above, h_b = _select_bucket(hist, sufs, kneed)

      def d_body(i, carry):
        aoff, coff = carry
        keys, ivs, ams, cms, cas, ccs = [], [], [], [], [], []
        for j in range(UN):
          b = (i * UN + j) * 16
          key = buf_k[pl.ds(b, 16)]
          iv = buf_i[pl.ds(b, 16)]
          lvm = (_iota16() + b) < count
          d = _lsr(key, shift) & jnp.int32(0xFF)
          am = (d > bstar) & lvm
          cm = (d == bstar) & lvm
          keys.append(key)
          ivs.append(iv)
          ams.append(am)
          cms.append(cm)
          cas.append(jnp.sum(am.astype(jnp.int32)))
          ccs.append(jnp.sum(cm.astype(jnp.int32)))
        for j in range(UN):
          plsc.store_compressed(acc_k.at[pl.ds(aoff, 16)], keys[j],
                                mask=ams[j])
          plsc.store_compressed(acc_i.at[pl.ds(aoff, 16)], ivs[j],
                                mask=ams[j])
          plsc.store_compressed(buf_k.at[pl.ds(coff, 16)], keys[j],
                                mask=cms[j])
          plsc.store_compressed(buf_i.at[pl.ds(coff, 16)], ivs[j],
                                mask=cms[j])
          aoff = aoff + cas[j]
          coff = coff + ccs[j]
        return (aoff, coff)

      aoff, _u = lax.fori_loop(0, nv4, d_body, (aoff, jnp.int32(0)))
      return aoff, kneed - g_above, h_b

    aoff, kneed, count = level(16, aoff, kneed, count)
    aoff, kneed, count = level(8, aoff, kneed, count)
    aoff, kneed, count = level(0, aoff, kneed, count)

    def e_body(i, aoff):
      base = i * 16
      key = buf_k[pl.ds(base, 16)]
      iv = buf_i[pl.ds(base, 16)]
      m = (_iota16() + base) < kneed
      plsc.store_compressed(acc_k.at[pl.ds(aoff, 16)], key, mask=m)
      plsc.store_compressed(acc_i.at[pl.ds(aoff, 16)], iv, mask=m)
      return aoff + jnp.sum(m.astype(jnp.int32))

    lax.fori_loop(0, (kneed + 15) >> 4, e_body, aoff)

    def sort_pass(src_k, src_i, dst_k, dst_i, shift, invert):
      _clear(hist4, UN * 256)
      qoffs = [offs_a, offs_b, offs_c, offs_d]

      @plsc.parallel_loop(0, KV, 1, unroll=UN)
      def h_body(i):
        k_ = src_k[pl.ds(i * 16, 16)]
        if invert:
          k_ = ~k_
        d = _lsr(k_, shift) & jnp.int32(0xFF)
        occ, last = plsc.scan_count(d)
        plsc.addupdate_scatter(hist4, [d + (i >> 5) * 256], occ, mask=last)

      run = jnp.int32(0)
      for v in range(16):
        hq = [hist4[pl.ds(q * 256 + v * 16, 16)] for q in range(4)]
        tot = (hq[0] + hq[1]) + (hq[2] + hq[3])
        c = plsc.cumsum(tot)
        ex = c - tot + run
        qoffs[0][pl.ds(v * 16, 16)] = ex
        ex = ex + hq[0]
        qoffs[1][pl.ds(v * 16, 16)] = ex
        ex = ex + hq[1]
        qoffs[2][pl.ds(v * 16, 16)] = ex
        ex = ex + hq[2]
        qoffs[3][pl.ds(v * 16, 16)] = ex
        run = run + jnp.max(c)

      def s_body(i, _):
        for q in range(4):
          b = (q * 32 + i) * 16
          k_ = src_k[pl.ds(b, 16)]
          iv = src_i[pl.ds(b, 16)]
          if invert:
            k_ = ~k_
          d = _lsr(k_, shift) & jnp.int32(0xFF)
          occ, last = plsc.scan_count(d)
          basev = plsc.load_gather(qoffs[q], [d])
          posn = basev + occ - 1
          plsc.store_scatter(dst_k, [posn], k_)
          plsc.store_scatter(dst_i, [posn], iv)
          plsc.addupdate_scatter(qoffs[q], [d], occ, mask=last)
        return 0

      lax.fori_loop(0, KV // 4, s_body, 0)

    sort_pass(acc_k, acc_i, tmp_k, tmp_i, 0, True)
    sort_pass(tmp_k, tmp_i, acc_k, acc_i, 8, False)
    sort_pass(acc_k, acc_i, tmp_k, tmp_i, 16, False)
    sort_pass(tmp_k, tmp_i, acc_k, acc_i, 24, False)

    @plsc.parallel_loop(0, KV, 1, unroll=UN)
    def g_body(i):
      b = i * 16
      sk = acc_k[pl.ds(b, 16)]
      v = _from_key(~sk)
      vals_out[pl.ds(b, 16)] = v
      iv = acc_i[pl.ds(b, 16)]
      idx_out[pl.ds(b, 16)] = jnp.where(
          v > jnp.float32(NEG_HALF), iv, jnp.int32(-1))
    pltpu.sync_copy(vals_out, vals_hbm.at[row])
    pltpu.sync_copy(idx_out, idx_hbm.at[row])
    return 0

  lax.fori_loop(0, 4, row_body, 0)


@jax.jit
def _sc_topk(scores, lengths):
  mesh = plsc.VectorSubcoreMesh(core_axis_name="c", subcore_axis_name="s")
  f = pl.kernel(
      _sc_body,
      out_type=(jax.ShapeDtypeStruct((B, K), jnp.float32),
                jax.ShapeDtypeStruct((B, K), jnp.int32)),
      mesh=mesh,
      compiler_params=pltpu.CompilerParams(needs_layout_passes=False),
      scratch_types=[
          pltpu.VMEM((N,), jnp.float32),
          pltpu.VMEM((N + 16,), jnp.int32),
          pltpu.VMEM((N + 16,), jnp.int32),
          pltpu.VMEM((K + 16,), jnp.int32),
          pltpu.VMEM((K + 16,), jnp.int32),
          pltpu.VMEM((K,), jnp.int32),
          pltpu.VMEM((K,), jnp.int32),
          pltpu.VMEM((UN * 256,), jnp.int32),
          pltpu.VMEM((4096,), jnp.int32),
          pltpu.VMEM((256,), jnp.int32),
          pltpu.VMEM((256,), jnp.int32),
          pltpu.VMEM((256,), jnp.int32),
          pltpu.VMEM((256,), jnp.int32),
          pltpu.VMEM((256,), jnp.int32),
          pltpu.VMEM((256,), jnp.int32),
          pltpu.VMEM((B,), jnp.int32),
          pltpu.VMEM((K,), jnp.float32),
          pltpu.VMEM((K,), jnp.int32),
          pltpu.SemaphoreType.DMA,
      ],
  )
  return f(scores, lengths)


def kernel(scores, lengths, k):
  del k
  return _sc_topk(scores, lengths)


# speedup vs baseline: 1.2249x; 1.0041x over previous
import jax
import jax.numpy as jnp
from jax import lax
from jax.experimental import pallas as pl
from jax.experimental.pallas import tpu as pltpu
from jax.experimental.pallas import tpu_sc as plsc

B = 128
N = 32768
K = 2048
NV = N // 16
KV = K // 16
UN = 4
MININT = -2147483648
NEG = -3.4028234663852886e38
NEG_HALF = -1.7014117331926443e38
NEG_KEY = 8388608


def _iota16():
  return lax.broadcasted_iota(jnp.int32, (16,), 0)


def _lsr(x, n):
  return lax.shift_right_logical(x, lax.full(x.shape, n, x.dtype))


def _to_key(x):
  s = plsc.bitcast(x, jnp.int32)
  return jnp.where(s >= 0, s | jnp.int32(MININT), ~s)


def _from_key(key):
  bits = jnp.where(key < 0, key & jnp.int32(0x7FFFFFFF), ~key)
  return plsc.bitcast(bits, jnp.float32)


def _clear(ref, nbins):
  for v in range(nbins // 16):
    ref[pl.ds(v * 16, 16)] = jnp.zeros((16,), jnp.int32)


def _merge_hists(hist4, hist):
  for v in range(16):
    s = hist4[pl.ds(v * 16, 16)]
    for j in range(1, UN):
      s = s + hist4[pl.ds(j * 256 + v * 16, 16)]
    hist[pl.ds(v * 16, 16)] = s


def _select_bucket(hist, sufs, kneed):
  run = jnp.int32(0)
  bstar = jnp.int32(-1)
  for v in range(15, -1, -1):
    x = hist[pl.ds(v * 16, 16)]
    c = plsc.cumsum(x)
    tot = jnp.max(c)
    suf = (tot - c) + x + run
    sufs[pl.ds(v * 16, 16)] = suf
    bid = _iota16() + jnp.int32(v * 16)
    cand = jnp.where(suf >= kneed, bid, jnp.int32(-1))
    bstar = jnp.maximum(bstar, jnp.max(cand))
    run = run + tot
  bvec = jnp.broadcast_to(bstar, (16,))
  s_b = jnp.max(plsc.load_gather(sufs, [bvec]))
  h_b = jnp.max(plsc.load_gather(hist, [bvec]))
  return bstar, s_b - h_b, h_b


def _sc_body(scores_hbm, lengths_hbm, vals_hbm, idx_hbm,
             row_f, buf_k, buf_i, acc_k, acc_i, tmp_k, tmp_i,
             hist4, hists16, hist, sufs, offs_a, offs_b, offs_c, offs_d,
             len_v, vals_out, idx_out, dma_sem, out_sem):
  cid = lax.axis_index("c")
  sid = lax.axis_index("s")
  wid = cid * 16 + sid
  pltpu.sync_copy(lengths_hbm, len_v)
  pltpu.async_copy(scores_hbm.at[wid], row_f, dma_sem)

  def row_body(r, _):
    row = wid + 32 * r
    grp16 = (row >> 4) << 4
    lv = len_v[pl.ds(grp16, 16)]
    lane = row - grp16
    length = jnp.max(jnp.where(_iota16() == lane, lv, jnp.int32(0)))
    nvl4 = ((length + 15) >> 4).astype(jnp.int32)
    nvl4 = (nvl4 + (UN - 1)) >> 2
    tail_start = nvl4 * (UN * 16)
    tail_cnt = jnp.int32(N) - tail_start

    pltpu.make_async_copy(scores_hbm.at[row], row_f,
                          dma_sem).wait()

    def clr_body(i, _):
      for j in range(UN):
        hists16[pl.ds((i * UN + j) * 16, 16)] = jnp.zeros((16,), jnp.int32)
      return 0

    lax.fori_loop(0, 256 // UN, clr_body, 0)
    ones = jnp.ones((16,), jnp.int32)

    @plsc.parallel_loop(0, nvl4 * UN, 1, unroll=UN)
    def a_body(i):
      b = i * 16
      key = _to_key(row_f[pl.ds(b, 16)])
      pos = _iota16() + b
      key = jnp.where(pos < length, key, jnp.int32(NEG_KEY))
      d = _lsr(key, 24)
      addr = (d << 4) | _iota16()
      plsc.addupdate_scatter(hists16, [addr], ones)
    lane16 = _iota16() * 16
    for v in range(16):
      tot = plsc.load_gather(hists16, [lane16 + jnp.int32(v * 256)])
      for l in range(1, 16):
        tot = tot + plsc.load_gather(hists16,
                                     [lane16 + jnp.int32(v * 256 + l)])
      hist[pl.ds(v * 16, 16)] = tot
    h0 = hist[pl.ds(0, 16)]
    hist[pl.ds(0, 16)] = h0 + jnp.where(_iota16() == 0, tail_cnt,
                                        jnp.int32(0))

    kneed = jnp.int32(K)
    bstar, g_above, h_b = _select_bucket(hist, sufs, kneed)

    def c_body(i, carry):
      aoff, coff = carry
      keys, ams, cms, cas, ccs, poss = [], [], [], [], [], []
      for j in range(UN):
        b = (i * UN + j) * 16
        key = _to_key(row_f[pl.ds(b, 16)])
        pos = _iota16() + b
        key = jnp.where(pos < length, key, jnp.int32(NEG_KEY))
        d = _lsr(key, 24)
        am = d > bstar
        cm = d == bstar
        keys.append(key)
        poss.append(pos)
        ams.append(am)
        cms.append(cm)
        cas.append(jnp.sum(am.astype(jnp.int32)))
        ccs.append(jnp.sum(cm.astype(jnp.int32)))
      for j in range(UN):
        plsc.store_compressed(acc_k.at[pl.ds(aoff, 16)], keys[j],
                              mask=ams[j])
        plsc.store_compressed(acc_i.at[pl.ds(aoff, 16)], poss[j],
                              mask=ams[j])
        plsc.store_compressed(buf_k.at[pl.ds(coff, 16)], keys[j],
                              mask=cms[j])
        plsc.store_compressed(buf_i.at[pl.ds(coff, 16)], poss[j],
                              mask=cms[j])
        aoff = aoff + cas[j]
        coff = coff + ccs[j]
      return (aoff, coff)

    aoff, coff = lax.fori_loop(0, nvl4, c_body,
                               (jnp.int32(0), jnp.int32(0)))

    kneed = kneed - g_above

    tail_stop = jnp.minimum(jnp.int32(NV),
                            (tail_start >> 4) + ((kneed + 15) >> 4))
    ncand = jnp.where(bstar == 0, coff + (tail_stop << 4) - tail_start, h_b)

    @pl.when(bstar == 0)
    def _():
      all_true = jnp.ones((16,), jnp.bool_)
      negs = jnp.full((16,), NEG_KEY, jnp.int32)

      def t_body(v, coff_t):
        plsc.store_compressed(buf_k.at[pl.ds(coff_t, 16)], negs,
                              mask=all_true)
        plsc.store_compressed(buf_i.at[pl.ds(coff_t, 16)],
                              _iota16() + v * 16, mask=all_true)
        return coff_t + 16

      lax.fori_loop(tail_start >> 4, tail_stop, t_body, coff)

    @pl.when(r < 3)
    def _():
      pltpu.async_copy(scores_hbm.at[row + 32], row_f,
                       dma_sem)

    count = ncand

    def level(shift, aoff, kneed, count):
      _clear(hist4, UN * 256)
      nv4 = (count + (UN * 16 - 1)) >> 6

      @plsc.parallel_loop(0, nv4 * UN, 1, unroll=UN)
      def h_body(i):
        b = i * 16
        key = buf_k[pl.ds(b, 16)]
        lvm = (_iota16() + b) < count
        d = _lsr(key, shift) & jnp.int32(0xFF)
        occ, last = plsc.scan_count(d, mask=lvm)
        plsc.addupdate_scatter(hist4, [d + (i & 3) * 256], occ, mask=last)

      _merge_hists(hist4, hist)
      bstar, g_above, h_b = _select_bucket(hist, sufs, kneed)

      def d_body(i, carry):
        aoff, coff = carry
        keys, ivs, ams, cms, cas, ccs = [], [], [], [], [], []
        for j in range(UN):
          b = (i * UN + j) * 16
          key = buf_k[pl.ds(b, 16)]
          iv = buf_i[pl.ds(b, 16)]
          lvm = (_iota16() + b) < count
          d = _lsr(key, shift) & jnp.int32(0xFF)
          am = (d > bstar) & lvm
          cm = (d == bstar) & lvm
          keys.append(key)
          ivs.append(iv)
          ams.append(am)
          cms.append(cm)
          cas.append(jnp.sum(am.astype(jnp.int32)))
          ccs.append(jnp.sum(cm.astype(jnp.int32)))
        for j in range(UN):
          plsc.store_compressed(acc_k.at[pl.ds(aoff, 16)], keys[j],
                                mask=ams[j])
          plsc.store_compressed(acc_i.at[pl.ds(aoff, 16)], ivs[j],
                                mask=ams[j])
          plsc.store_compressed(buf_k.at[pl.ds(coff, 16)], keys[j],
                                mask=cms[j])
          plsc.store_compressed(buf_i.at[pl.ds(coff, 16)], ivs[j],
                                mask=cms[j])
          aoff = aoff + cas[j]
          coff = coff + ccs[j]
        return (aoff, coff)

      aoff, _u = lax.fori_loop(0, nv4, d_body, (aoff, jnp.int32(0)))
      return aoff, kneed - g_above, h_b

    aoff, kneed, count = level(16, aoff, kneed, count)
    aoff, kneed, count = level(8, aoff, kneed, count)
    aoff, kneed, count = level(0, aoff, kneed, count)

    def e_body(i, aoff):
      base = i * 16
      key = buf_k[pl.ds(base, 16)]
      iv = buf_i[pl.ds(base, 16)]
      m = (_iota16() + base) < kneed
      plsc.store_compressed(acc_k.at[pl.ds(aoff, 16)], key, mask=m)
      plsc.store_compressed(acc_i.at[pl.ds(aoff, 16)], iv, mask=m)
      return aoff + jnp.sum(m.astype(jnp.int32))

    lax.fori_loop(0, (kneed + 15) >> 4, e_body, aoff)

    def sort_pass(src_k, src_i, dst_k, dst_i, shift, invert):
      _clear(hist4, UN * 256)
      qoffs = [offs_a, offs_b, offs_c, offs_d]

      @plsc.parallel_loop(0, KV, 1, unroll=UN)
      def h_body(i):
        k_ = src_k[pl.ds(i * 16, 16)]
        if invert:
          k_ = ~k_
        d = _lsr(k_, shift) & jnp.int32(0xFF)
        occ, last = plsc.scan_count(d)
        plsc.addupdate_scatter(hist4, [d + (i >> 5) * 256], occ, mask=last)

      run = jnp.int32(0)
      for v in range(16):
        hq = [hist4[pl.ds(q * 256 + v * 16, 16)] for q in range(4)]
        tot = (hq[0] + hq[1]) + (hq[2] + hq[3])
        c = plsc.cumsum(tot)
        ex = c - tot + run
        qoffs[0][pl.ds(v * 16, 16)] = ex
        ex = ex + hq[0]
        qoffs[1][pl.ds(v * 16, 16)] = ex
        ex = ex + hq[1]
        qoffs[2][pl.ds(v * 16, 16)] = ex
        ex = ex + hq[2]
        qoffs[3][pl.ds(v * 16, 16)] = ex
        run = run + jnp.max(c)

      def s_body(i, _):
        for q in range(4):
          b = (q * 32 + i) * 16
          k_ = src_k[pl.ds(b, 16)]
          iv = src_i[pl.ds(b, 16)]
          if invert:
            k_ = ~k_
          d = _lsr(k_, shift) & jnp.int32(0xFF)
          occ, last = plsc.scan_count(d)
          basev = plsc.load_gather(qoffs[q], [d])
          posn = basev + occ - 1
          plsc.store_scatter(dst_k, [posn], k_)
          plsc.store_scatter(dst_i, [posn], iv)
          plsc.addupdate_scatter(qoffs[q], [d], occ, mask=last)
        return 0

      lax.fori_loop(0, KV // 4, s_body, 0)

    sort_pass(acc_k, acc_i, tmp_k, tmp_i, 0, True)
    sort_pass(tmp_k, tmp_i, acc_k, acc_i, 8, False)
    sort_pass(acc_k, acc_i, tmp_k, tmp_i, 16, False)
    sort_pass(tmp_k, tmp_i, acc_k, acc_i, 24, False)

    @pl.when(r > 0)
    def _():
      pltpu.make_async_copy(vals_out, vals_hbm.at[row], out_sem).wait()
      pltpu.make_async_copy(idx_out, idx_hbm.at[row], out_sem).wait()

    @plsc.parallel_loop(0, KV, 1, unroll=UN)
    def g_body(i):
      b = i * 16
      sk = acc_k[pl.ds(b, 16)]
      v = _from_key(~sk)
      vals_out[pl.ds(b, 16)] = v
      iv = acc_i[pl.ds(b, 16)]
      idx_out[pl.ds(b, 16)] = jnp.where(
          v > jnp.float32(NEG_HALF), iv, jnp.int32(-1))
    pltpu.async_copy(vals_out, vals_hbm.at[row], out_sem)
    pltpu.async_copy(idx_out, idx_hbm.at[row], out_sem)
    return 0

  lax.fori_loop(0, 4, row_body, 0)
  pltpu.make_async_copy(vals_out, vals_hbm.at[0], out_sem).wait()
  pltpu.make_async_copy(idx_out, idx_hbm.at[0], out_sem).wait()


@jax.jit
def _sc_topk(scores, lengths):
  mesh = plsc.VectorSubcoreMesh(core_axis_name="c", subcore_axis_name="s")
  f = pl.kernel(
      _sc_body,
      out_type=(jax.ShapeDtypeStruct((B, K), jnp.float32),
                jax.ShapeDtypeStruct((B, K), jnp.int32)),
      mesh=mesh,
      compiler_params=pltpu.CompilerParams(needs_layout_passes=False),
      scratch_types=[
          pltpu.VMEM((N,), jnp.float32),
          pltpu.VMEM((N + 16,), jnp.int32),
          pltpu.VMEM((N + 16,), jnp.int32),
          pltpu.VMEM((K + 16,), jnp.int32),
          pltpu.VMEM((K + 16,), jnp.int32),
          pltpu.VMEM((K,), jnp.int32),
          pltpu.VMEM((K,), jnp.int32),
          pltpu.VMEM((UN * 256,), jnp.int32),
          pltpu.VMEM((4096,), jnp.int32),
          pltpu.VMEM((256,), jnp.int32),
          pltpu.VMEM((256,), jnp.int32),
          pltpu.VMEM((256,), jnp.int32),
          pltpu.VMEM((256,), jnp.int32),
          pltpu.VMEM((256,), jnp.int32),
          pltpu.VMEM((256,), jnp.int32),
          pltpu.VMEM((B,), jnp.int32),
          pltpu.VMEM((K,), jnp.float32),
          pltpu.VMEM((K,), jnp.int32),
          pltpu.SemaphoreType.DMA,
          pltpu.SemaphoreType.DMA,
      ],
  )
  return f(scores, lengths)


def kernel(scores, lengths, k):
  del k
  return _sc_topk(scores, lengths)
